# baseline (speedup 1.0000x reference)
import functools
import jax
import jax.numpy as jnp
from jax.experimental import pallas as pl
from jax.experimental.pallas import tpu as pltpu

_BB = 64


def _blend_body(y_ref, t_ref, x_ref, o_ref):
    t = t_ref[...]

    def step(b, carry):
        mf = jnp.where(y_ref[b, 0] == 0, 1.0, 0.0)
        xb = x_ref[b]
        o_ref[b] = xb - (mf * t) * (xb + 0.75)
        return carry

    jax.lax.fori_loop(0, _BB, step, 0, unroll=8)


def kernel(x, y, template):
    B, C, S, _ = x.shape
    R = C * S
    x3 = x.reshape(B, R, S)
    t3 = jnp.tile(template, (C, 1))
    out = pl.pallas_call(
        _blend_body,
        grid=(B // _BB,),
        in_specs=[
            pl.BlockSpec((_BB, 1), lambda i: (i, 0), memory_space=pltpu.SMEM),
            pl.BlockSpec((R, S), lambda i: (0, 0)),
            pl.BlockSpec((_BB, R, S), lambda i: (i, 0, 0)),
        ],
        out_specs=pl.BlockSpec((_BB, R, S), lambda i: (i, 0, 0)),
        out_shape=jax.ShapeDtypeStruct((B, R, S), x.dtype),
    )(y, t3, x3)
    return (out.reshape(x.shape), y)
